# baseline (speedup 1.0000x reference)
import functools

import jax
import jax.numpy as jnp
from jax import lax
from jax.experimental import pallas as pl
from jax.experimental.pallas import tpu as pltpu
from jax.experimental.pallas import tpu_sc as plsc

B, T = 4, 8192
N_TOK = B * T
D_MODEL = 1024
SPD = 86400
PERIODS = (SPD, SPD // 2, SPD // 3, SPD // 4, SPD // 6)
NT = len(PERIODS)
SUB = 204
HEAD = 128
TAIL = SUB - HEAD
SEG = 256
K_TOT = NT * SEG
LANES = 16
NC, NS = 2, 16
NW = NC * NS
TOK_W = N_TOK // NW
CHUNK = 128
NCHUNK = TOK_W // CHUNK


def _mod_period(v, period):
    out = v
    k = period
    while k < SPD:
        out = out - jnp.where(v >= k, jnp.int32(period), jnp.int32(0))
        k += period
    return out


def _sc_gather_one(t_flat, table, tail, period):
    mesh = plsc.VectorSubcoreMesh(core_axis_name="c", subcore_axis_name="s")
    out_type = jax.ShapeDtypeStruct((N_TOK, SEG), jnp.float32)
    scratch = (
        [pltpu.VMEM((TOK_W,), jnp.int32),
         pltpu.VMEM((TOK_W,), jnp.int32)]
        + [pltpu.VMEM((CHUNK, HEAD), jnp.float32) for _ in range(4)]
        + [pltpu.SemaphoreType.DMA for _ in range(4)]
    )

    @functools.partial(pl.kernel, mesh=mesh, out_type=out_type,
                       scratch_types=scratch)
    def k(t_hbm, tbl, tl, out, tok_v, idx_v, bh0, bt0, bh1, bt1,
          smh0, smt0, smh1, smt1):
        wid = lax.axis_index("s") * NC + lax.axis_index("c")
        base = wid * TOK_W
        pltpu.sync_copy(t_hbm.at[pl.ds(base, TOK_W)], tok_v)

        def mod_body(c, carry):
            off = c * LANES
            idx_v[pl.ds(off, LANES)] = _mod_period(
                tok_v[pl.ds(off, LANES)], period)
            return carry

        lax.fori_loop(0, TOK_W // LANES, mod_body, 0)

        hcol = pl.ds(0, HEAD)
        tcol = pl.ds(HEAD, HEAD)

        def pair_body(c, carry):
            ch0 = c * 2
            ch1 = ch0 + 1
            ix0 = idx_v.at[pl.ds(ch0 * CHUNK, CHUNK)]
            ix1 = idx_v.at[pl.ds(ch1 * CHUNK, CHUNK)]
            h0 = pltpu.async_copy(tbl.at[ix0], bh0, smh0)
            t0 = pltpu.async_copy(tl.at[ix0], bt0, smt0)
            h1 = pltpu.async_copy(tbl.at[ix1], bh1, smh1)
            t1 = pltpu.async_copy(tl.at[ix1], bt1, smt1)
            r0 = pl.ds(base + ch0 * CHUNK, CHUNK)
            r1 = pl.ds(base + ch1 * CHUNK, CHUNK)
            h0.wait()
            pltpu.sync_copy(bh0, out.at[r0, hcol])
            t0.wait()
            pltpu.sync_copy(bt0, out.at[r0, tcol])
            h1.wait()
            pltpu.sync_copy(bh1, out.at[r1, hcol])
            t1.wait()
            pltpu.sync_copy(bt1, out.at[r1, tcol])
            return carry

        lax.fori_loop(0, NCHUNK // 2, pair_body, 0)

    return k(t_flat, table, tail)


def _tc_project(embs, w, bias):
    BM = 512
    ne = len(embs)

    def body(*refs):
        e_refs = refs[:ne]
        w_ref = refs[ne]
        b_ref = refs[ne + 1]
        o_ref = refs[ne + 2]
        acc = b_ref[...]
        for i, e_ref in enumerate(e_refs):
            acc = acc + jnp.dot(e_ref[...].astype(jnp.bfloat16),
                                w_ref[i * SEG:(i + 1) * SEG],
                                preferred_element_type=jnp.float32)
        o_ref[...] = acc

    in_specs = (
        [pl.BlockSpec((BM, SEG), lambda m: (m, 0)) for _ in range(ne)]
        + [pl.BlockSpec((K_TOT, D_MODEL), lambda m: (0, 0)),
           pl.BlockSpec((1, D_MODEL), lambda m: (0, 0))]
    )
    return pl.pallas_call(
        body,
        grid=(N_TOK // BM,),
        in_specs=in_specs,
        out_specs=pl.BlockSpec((BM, D_MODEL), lambda m: (m, 0)),
        out_shape=jax.ShapeDtypeStruct((N_TOK, D_MODEL), jnp.float32),
    )(*embs, w, bias)


def kernel(x, time_indices, table0, table1, table2, table3, table4, Wp, bp):
    del x
    t_flat = time_indices.reshape(N_TOK).astype(jnp.int32)
    tables = (table0, table1, table2, table3, table4)
    embs = []
    for i, tbl in enumerate(tables):
        head = tbl[:, :HEAD]
        tail = jnp.pad(tbl[:, HEAD:], ((0, 0), (0, HEAD - TAIL)))
        embs.append(_sc_gather_one(t_flat, head, tail, PERIODS[i]))
    zrows = jnp.zeros((SEG - SUB, D_MODEL), jnp.float32)
    w = jnp.concatenate(
        [p for i in range(NT)
         for p in (Wp[i * SUB:(i + 1) * SUB], zrows)]).astype(jnp.bfloat16)
    out = _tc_project(embs, w, bp.reshape(1, D_MODEL))
    return out.reshape(B, T, D_MODEL)


# speedup vs baseline: 1.1108x; 1.1108x over previous
import functools

import jax
import jax.numpy as jnp
from jax import lax
from jax.experimental import pallas as pl
from jax.experimental.pallas import tpu as pltpu
from jax.experimental.pallas import tpu_sc as plsc

B, T = 4, 8192
N_TOK = B * T
D_MODEL = 1024
SPD = 86400
PERIODS = (SPD, SPD // 2, SPD // 3, SPD // 4, SPD // 6)
NT = len(PERIODS)
SUB = 204
HEAD = 128
TAIL = SUB - HEAD
SEG = 256
K_TOT = NT * SEG
LANES = 16
NC, NS = 2, 16
NW = NC * NS
TOK_W = N_TOK // NW
CHUNK = 128
NCHUNK = TOK_W // CHUNK


def _mod_period(v, period):
    out = v
    k = period
    while k < SPD:
        out = out - jnp.where(v >= k, jnp.int32(period), jnp.int32(0))
        k += period
    return out


def _sc_gather_one(t_flat, table, tail, period):
    mesh = plsc.VectorSubcoreMesh(core_axis_name="c", subcore_axis_name="s")
    out_type = jax.ShapeDtypeStruct((N_TOK, SEG), jnp.float32)
    scratch = (
        [pltpu.VMEM((TOK_W,), jnp.int32),
         pltpu.VMEM((TOK_W,), jnp.int32)]
        + [pltpu.VMEM((CHUNK, HEAD), jnp.float32) for _ in range(4)]
        + [pltpu.SemaphoreType.DMA for _ in range(4)]
    )

    @functools.partial(pl.kernel, mesh=mesh, out_type=out_type,
                       scratch_types=scratch)
    def k(t_hbm, tbl, tl, out, tok_v, idx_v, bh0, bt0, bh1, bt1,
          smh0, smt0, smh1, smt1):
        wid = lax.axis_index("s") * NC + lax.axis_index("c")
        base = wid * TOK_W
        pltpu.sync_copy(t_hbm.at[pl.ds(base, TOK_W)], tok_v)

        def mod_body(c, carry):
            off = c * LANES
            idx_v[pl.ds(off, LANES)] = _mod_period(
                tok_v[pl.ds(off, LANES)], period)
            return carry

        lax.fori_loop(0, TOK_W // LANES, mod_body, 0)

        hcol = pl.ds(0, HEAD)
        tcol = pl.ds(HEAD, HEAD)

        def pair_body(c, carry):
            ch0 = c * 2
            ch1 = ch0 + 1
            ix0 = idx_v.at[pl.ds(ch0 * CHUNK, CHUNK)]
            ix1 = idx_v.at[pl.ds(ch1 * CHUNK, CHUNK)]
            h0 = pltpu.async_copy(tbl.at[ix0, pl.ds(0, HEAD)], bh0, smh0)
            t0 = pltpu.async_copy(tl.at[ix0], bt0, smt0)
            h1 = pltpu.async_copy(tbl.at[ix1, pl.ds(0, HEAD)], bh1, smh1)
            t1 = pltpu.async_copy(tl.at[ix1], bt1, smt1)
            r0 = pl.ds(base + ch0 * CHUNK, CHUNK)
            r1 = pl.ds(base + ch1 * CHUNK, CHUNK)
            h0.wait()
            pltpu.sync_copy(bh0, out.at[r0, hcol])
            t0.wait()
            pltpu.sync_copy(bt0, out.at[r0, tcol])
            h1.wait()
            pltpu.sync_copy(bh1, out.at[r1, hcol])
            t1.wait()
            pltpu.sync_copy(bt1, out.at[r1, tcol])
            return carry

        lax.fori_loop(0, NCHUNK // 2, pair_body, 0)

    return k(t_flat, table, tail)


def _tc_project(embs, w, bias):
    BM = 512
    ne = len(embs)

    def body(*refs):
        e_refs = refs[:ne]
        w_ref = refs[ne]
        b_ref = refs[ne + 1]
        o_ref = refs[ne + 2]
        acc = b_ref[...]
        for i, e_ref in enumerate(e_refs):
            acc = acc + jnp.dot(e_ref[...].astype(jnp.bfloat16),
                                w_ref[i * SEG:(i + 1) * SEG],
                                preferred_element_type=jnp.float32)
        o_ref[...] = acc

    in_specs = (
        [pl.BlockSpec((BM, SEG), lambda m: (m, 0)) for _ in range(ne)]
        + [pl.BlockSpec((K_TOT, D_MODEL), lambda m: (0, 0)),
           pl.BlockSpec((1, D_MODEL), lambda m: (0, 0))]
    )
    return pl.pallas_call(
        body,
        grid=(N_TOK // BM,),
        in_specs=in_specs,
        out_specs=pl.BlockSpec((BM, D_MODEL), lambda m: (m, 0)),
        out_shape=jax.ShapeDtypeStruct((N_TOK, D_MODEL), jnp.float32),
    )(*embs, w, bias)


def kernel(x, time_indices, table0, table1, table2, table3, table4, Wp, bp):
    del x
    t_flat = time_indices.reshape(N_TOK).astype(jnp.int32)
    tables = (table0, table1, table2, table3, table4)
    embs = []
    for i, tbl in enumerate(tables):
        tail = jnp.pad(tbl[:, HEAD:], ((0, 0), (0, HEAD - TAIL)))
        embs.append(_sc_gather_one(t_flat, tbl, tail, PERIODS[i]))
    zrows = jnp.zeros((SEG - SUB, D_MODEL), jnp.float32)
    w = jnp.concatenate(
        [p for i in range(NT)
         for p in (Wp[i * SUB:(i + 1) * SUB], zrows)]).astype(jnp.bfloat16)
    out = _tc_project(embs, w, bp.reshape(1, D_MODEL))
    return out.reshape(B, T, D_MODEL)


# speedup vs baseline: 1.1563x; 1.0410x over previous
import functools

import jax
import jax.numpy as jnp
from jax import lax
from jax.experimental import pallas as pl
from jax.experimental.pallas import tpu as pltpu
from jax.experimental.pallas import tpu_sc as plsc

B, T = 4, 8192
N_TOK = B * T
D_MODEL = 1024
SPD = 86400
PERIODS = (SPD, SPD // 2, SPD // 3, SPD // 4, SPD // 6)
NT = len(PERIODS)
SUB = 204
HEAD = 128
TAIL = SUB - HEAD
SEG = 256
K_TOT = NT * SEG
LANES = 16
NC, NS = 2, 16
NW = NC * NS
TOK_W = N_TOK // NW
CHUNK = 128
NCHUNK = TOK_W // CHUNK


def _mod_period(v, period):
    out = v
    k = period
    while k < SPD:
        out = out - jnp.where(v >= k, jnp.int32(period), jnp.int32(0))
        k += period
    return out


def _sc_gather_one(t_flat, table, tail, period):
    mesh = plsc.VectorSubcoreMesh(core_axis_name="c", subcore_axis_name="s")
    out_type = jax.ShapeDtypeStruct((N_TOK, SEG), jnp.float32)
    scratch = (
        [pltpu.VMEM((TOK_W,), jnp.int32),
         pltpu.VMEM((TOK_W,), jnp.int32)]
        + [pltpu.VMEM((CHUNK, HEAD), jnp.float32) for _ in range(4)]
        + [pltpu.SemaphoreType.DMA for _ in range(4)]
    )

    @functools.partial(pl.kernel, mesh=mesh, out_type=out_type,
                       scratch_types=scratch)
    def k(t_hbm, tbl, tl, out, tok_v, idx_v, bh0, bt0, bh1, bt1,
          smh0, smt0, smh1, smt1):
        wid = lax.axis_index("s") * NC + lax.axis_index("c")
        base = wid * TOK_W
        pltpu.sync_copy(t_hbm.at[pl.ds(base, TOK_W)], tok_v)

        def mod_body(c, carry):
            off = c * LANES
            idx_v[pl.ds(off, LANES)] = _mod_period(
                tok_v[pl.ds(off, LANES)], period)
            return carry

        lax.fori_loop(0, TOK_W // LANES, mod_body, 0)

        hcol = pl.ds(0, HEAD)
        tcol = pl.ds(HEAD, HEAD)

        def pair_body(c, carry):
            ch0 = c * 2
            ch1 = ch0 + 1
            ix0 = idx_v.at[pl.ds(ch0 * CHUNK, CHUNK)]
            ix1 = idx_v.at[pl.ds(ch1 * CHUNK, CHUNK)]
            h0 = pltpu.async_copy(tbl.at[ix0, pl.ds(0, HEAD)], bh0, smh0)
            t0 = pltpu.async_copy(tl.at[ix0], bt0, smt0)
            h1 = pltpu.async_copy(tbl.at[ix1, pl.ds(0, HEAD)], bh1, smh1)
            t1 = pltpu.async_copy(tl.at[ix1], bt1, smt1)
            r0 = pl.ds(base + ch0 * CHUNK, CHUNK)
            r1 = pl.ds(base + ch1 * CHUNK, CHUNK)
            h0.wait()
            pltpu.sync_copy(bh0, out.at[r0, hcol])
            t0.wait()
            pltpu.sync_copy(bt0, out.at[r0, tcol])
            h1.wait()
            pltpu.sync_copy(bh1, out.at[r1, hcol])
            t1.wait()
            pltpu.sync_copy(bt1, out.at[r1, tcol])
            return carry

        lax.fori_loop(0, NCHUNK // 2, pair_body, 0)

    return k(t_flat, table, tail)


def _tc_project(embs, w, bias):
    BM = 1024
    ne = len(embs)

    def body(*refs):
        e_refs = refs[:ne]
        w_ref = refs[ne]
        b_ref = refs[ne + 1]
        o_ref = refs[ne + 2]
        acc = b_ref[...]
        for i, e_ref in enumerate(e_refs):
            acc = acc + jnp.dot(e_ref[...].astype(jnp.bfloat16),
                                w_ref[i * SEG:(i + 1) * SEG],
                                preferred_element_type=jnp.float32)
        o_ref[...] = acc

    in_specs = (
        [pl.BlockSpec((BM, SEG), lambda m: (m, 0)) for _ in range(ne)]
        + [pl.BlockSpec((K_TOT, D_MODEL), lambda m: (0, 0)),
           pl.BlockSpec((1, D_MODEL), lambda m: (0, 0))]
    )
    return pl.pallas_call(
        body,
        grid=(N_TOK // BM,),
        in_specs=in_specs,
        out_specs=pl.BlockSpec((BM, D_MODEL), lambda m: (m, 0)),
        out_shape=jax.ShapeDtypeStruct((N_TOK, D_MODEL), jnp.float32),
    )(*embs, w, bias)


def kernel(x, time_indices, table0, table1, table2, table3, table4, Wp, bp):
    del x
    t_flat = time_indices.reshape(N_TOK).astype(jnp.int32)
    tables = (table0, table1, table2, table3, table4)
    embs = []
    for i, tbl in enumerate(tables):
        tail = jnp.pad(tbl[:, HEAD:], ((0, 0), (0, HEAD - TAIL)))
        embs.append(_sc_gather_one(t_flat, tbl, tail, PERIODS[i]))
    zrows = jnp.zeros((SEG - SUB, D_MODEL), jnp.float32)
    w = jnp.concatenate(
        [p for i in range(NT)
         for p in (Wp[i * SUB:(i + 1) * SUB], zrows)]).astype(jnp.bfloat16)
    out = _tc_project(embs, w, bp.reshape(1, D_MODEL))
    return out.reshape(B, T, D_MODEL)
